# baseline (speedup 1.0000x reference)
import functools

import jax
import jax.numpy as jnp
from jax.experimental import pallas as pl

_Q = 1024
_D = 512
_BQ = 1024
_SPLIT = 2560
_DEF = jax.lax.Precision.DEFAULT


def _sim(q, p):
    return jax.lax.dot_general(
        q, p, (((1,), (1,)), ((), ())), preferred_element_type=jnp.float32,
        precision=_DEF)


def _wsum(e, p):
    return jax.lax.dot_general(
        e, p, (((1,), (0,)), ((), ())), preferred_element_type=jnp.float32,
        precision=_DEF)


def _retrieve(qb, p):
    sim = _sim(qb, p)
    m = jnp.max(sim, axis=-1, keepdims=True)
    e = jnp.exp(sim - m)
    s = jnp.sum(e, axis=-1, keepdims=True)
    return _wsum(e.astype(jnp.bfloat16), p) * (1.0 / s)


def _retrieve_split(qb, p1, p2):
    sim1 = _sim(qb, p1)
    sim2 = _sim(qb, p2)
    m1 = jnp.max(sim1, axis=-1, keepdims=True)
    m2 = jnp.max(sim2, axis=-1, keepdims=True)
    m = jnp.maximum(m1, m2)
    e1 = jnp.exp(sim1 - m)
    e2 = jnp.exp(sim2 - m)
    s = (jnp.sum(e1, axis=-1, keepdims=True)
         + jnp.sum(e2, axis=-1, keepdims=True))
    num = (_wsum(e1.astype(jnp.bfloat16), p1)
           + _wsum(e2.astype(jnp.bfloat16), p2))
    return num * (1.0 / s)


def _body(qb_ref, pg_ref, pa_ref, pb_ref, w1_ref, b1_ref, w2t_ref, b2_ref,
          o_ref):
    qb = qb_ref[...].astype(jnp.bfloat16)
    pg1 = pg_ref[0:_SPLIT, :].astype(jnp.bfloat16)
    pg2 = pg_ref[_SPLIT:, :].astype(jnp.bfloat16)
    rg = _retrieve_split(qb, pg1, pg2)
    ra = _retrieve(qb, pa_ref[...].astype(jnp.bfloat16))
    rb = _retrieve(qb, pb_ref[...].astype(jnp.bfloat16))
    cr = 0.5 * (ra + rb)

    comb = jnp.concatenate([cr, rg], axis=-1)
    h = jax.lax.dot_general(
        comb.astype(jnp.bfloat16), w1_ref[...].astype(jnp.bfloat16),
        (((1,), (0,)), ((), ())),
        preferred_element_type=jnp.float32, precision=_DEF) + b1_ref[...]
    h = 0.5 * h * (1.0 + jax.lax.erf(h * 0.7071067811865476))
    logit = jnp.sum(h * w2t_ref[...], axis=-1, keepdims=True) + b2_ref[...]
    gate = jax.nn.sigmoid(logit)
    o_ref[...] = gate * cr + (1.0 - gate) * rg


@functools.partial(jax.jit, static_argnames=())
def kernel(query, global_patterns, classA_patterns, classB_patterns,
           W1, b1, W2, b2):
    kg = global_patterns.shape[0]
    kc = classA_patterns.shape[0]
    grid = (_Q // _BQ,)
    out = pl.pallas_call(
        _body,
        grid=grid,
        in_specs=[
            pl.BlockSpec((_BQ, _D), lambda i: (i, 0)),
            pl.BlockSpec((kg, _D), lambda i: (0, 0)),
            pl.BlockSpec((kc, _D), lambda i: (0, 0)),
            pl.BlockSpec((kc, _D), lambda i: (0, 0)),
            pl.BlockSpec((2 * _D, 64), lambda i: (0, 0)),
            pl.BlockSpec((1, 64), lambda i: (0, 0)),
            pl.BlockSpec((1, 64), lambda i: (0, 0)),
            pl.BlockSpec((1, 1), lambda i: (0, 0)),
        ],
        out_specs=pl.BlockSpec((_BQ, _D), lambda i: (i, 0)),
        out_shape=jax.ShapeDtypeStruct((_Q, _D), jnp.float32),
    )(query, global_patterns, classA_patterns, classB_patterns,
      W1, b1.reshape(1, 64), W2.reshape(1, 64), b2.reshape(1, 1))
    return out


# speedup vs baseline: 1.0639x; 1.0639x over previous
import functools

import jax
import jax.numpy as jnp
from jax.experimental import pallas as pl

_Q = 1024
_D = 512
_BQ = 1024
_DEF = jax.lax.Precision.DEFAULT


def _retrieve(qb, p):
    sim = jax.lax.dot_general(
        qb, p, (((1,), (1,)), ((), ())), preferred_element_type=jnp.float32,
        precision=_DEF)
    m = jnp.max(sim, axis=-1, keepdims=True)
    e = jnp.exp(sim - m)
    s = jnp.sum(e, axis=-1, keepdims=True)
    num = jax.lax.dot_general(
        e.astype(jnp.bfloat16), p, (((1,), (0,)), ((), ())),
        preferred_element_type=jnp.float32, precision=_DEF)
    return num * (1.0 / s)


def _body(qb_ref, pg_ref, pa_ref, pb_ref, w1_ref, b1_ref, w2t_ref, b2_ref,
          o_ref):
    qb = qb_ref[...].astype(jnp.bfloat16)
    rg = _retrieve(qb, pg_ref[...].astype(jnp.bfloat16))
    ra = _retrieve(qb, pa_ref[...].astype(jnp.bfloat16))
    rb = _retrieve(qb, pb_ref[...].astype(jnp.bfloat16))
    cr = 0.5 * (ra + rb)

    comb = jnp.concatenate([cr, rg], axis=-1)
    h = jax.lax.dot_general(
        comb.astype(jnp.bfloat16), w1_ref[...].astype(jnp.bfloat16),
        (((1,), (0,)), ((), ())),
        preferred_element_type=jnp.float32, precision=_DEF) + b1_ref[...]
    h = 0.5 * h * (1.0 + jax.lax.erf(h * 0.7071067811865476))
    logit = jnp.sum(h * w2t_ref[...], axis=-1, keepdims=True) + b2_ref[...]
    gate = jax.nn.sigmoid(logit)
    o_ref[...] = gate * cr + (1.0 - gate) * rg


@functools.partial(jax.jit, static_argnames=())
def kernel(query, global_patterns, classA_patterns, classB_patterns,
           W1, b1, W2, b2):
    kg = global_patterns.shape[0]
    kc = classA_patterns.shape[0]
    grid = (_Q // _BQ,)
    out = pl.pallas_call(
        _body,
        grid=grid,
        in_specs=[
            pl.BlockSpec((_BQ, _D), lambda i: (i, 0)),
            pl.BlockSpec((kg, _D), lambda i: (0, 0)),
            pl.BlockSpec((kc, _D), lambda i: (0, 0)),
            pl.BlockSpec((kc, _D), lambda i: (0, 0)),
            pl.BlockSpec((2 * _D, 64), lambda i: (0, 0)),
            pl.BlockSpec((1, 64), lambda i: (0, 0)),
            pl.BlockSpec((1, 64), lambda i: (0, 0)),
            pl.BlockSpec((1, 1), lambda i: (0, 0)),
        ],
        out_specs=pl.BlockSpec((_BQ, _D), lambda i: (i, 0)),
        out_shape=jax.ShapeDtypeStruct((_Q, _D), jnp.float32),
    )(query, global_patterns, classA_patterns, classB_patterns,
      W1, b1.reshape(1, 64), W2.reshape(1, 64), b2.reshape(1, 1))
    return out


# speedup vs baseline: 1.0728x; 1.0084x over previous
import functools

import jax
import jax.numpy as jnp
from jax.experimental import pallas as pl

_Q = 1024
_D = 512
_BQ = 1024
_DEF = jax.lax.Precision.DEFAULT


def _retrieve(qb, p):
    sim = jax.lax.dot_general(
        qb, p, (((1,), (1,)), ((), ())), preferred_element_type=jnp.float32,
        precision=_DEF)
    m = jnp.max(sim, axis=-1, keepdims=True)
    e = jnp.exp((sim - m).astype(jnp.bfloat16))
    s = jnp.sum(e, axis=-1, keepdims=True, dtype=jnp.float32)
    num = jax.lax.dot_general(
        e, p, (((1,), (0,)), ((), ())),
        preferred_element_type=jnp.float32, precision=_DEF)
    return num * (1.0 / s)


def _body(qb_ref, pg_ref, pa_ref, pb_ref, w1_ref, b1_ref, w2t_ref, b2_ref,
          o_ref):
    qb = qb_ref[...].astype(jnp.bfloat16)
    rg = _retrieve(qb, pg_ref[...].astype(jnp.bfloat16))
    ra = _retrieve(qb, pa_ref[...].astype(jnp.bfloat16))
    rb = _retrieve(qb, pb_ref[...].astype(jnp.bfloat16))
    cr = 0.5 * (ra + rb)

    comb = jnp.concatenate([cr, rg], axis=-1)
    h = jax.lax.dot_general(
        comb.astype(jnp.bfloat16), w1_ref[...].astype(jnp.bfloat16),
        (((1,), (0,)), ((), ())),
        preferred_element_type=jnp.float32, precision=_DEF) + b1_ref[...]
    h = 0.5 * h * (1.0 + jax.lax.erf(h * 0.7071067811865476))
    logit = jnp.sum(h * w2t_ref[...], axis=-1, keepdims=True) + b2_ref[...]
    gate = jax.nn.sigmoid(logit)
    o_ref[...] = gate * cr + (1.0 - gate) * rg


@functools.partial(jax.jit, static_argnames=())
def kernel(query, global_patterns, classA_patterns, classB_patterns,
           W1, b1, W2, b2):
    kg = global_patterns.shape[0]
    kc = classA_patterns.shape[0]
    grid = (_Q // _BQ,)
    out = pl.pallas_call(
        _body,
        grid=grid,
        in_specs=[
            pl.BlockSpec((_BQ, _D), lambda i: (i, 0)),
            pl.BlockSpec((kg, _D), lambda i: (0, 0)),
            pl.BlockSpec((kc, _D), lambda i: (0, 0)),
            pl.BlockSpec((kc, _D), lambda i: (0, 0)),
            pl.BlockSpec((2 * _D, 64), lambda i: (0, 0)),
            pl.BlockSpec((1, 64), lambda i: (0, 0)),
            pl.BlockSpec((1, 64), lambda i: (0, 0)),
            pl.BlockSpec((1, 1), lambda i: (0, 0)),
        ],
        out_specs=pl.BlockSpec((_BQ, _D), lambda i: (i, 0)),
        out_shape=jax.ShapeDtypeStruct((_Q, _D), jnp.float32),
    )(query, global_patterns, classA_patterns, classB_patterns,
      W1, b1.reshape(1, 64), W2.reshape(1, 64), b2.reshape(1, 1))
    return out


# speedup vs baseline: 1.1046x; 1.0296x over previous
import functools

import jax
import jax.numpy as jnp
from jax.experimental import pallas as pl

_Q = 1024
_D = 512
_BQ = 1024
_DEF = jax.lax.Precision.DEFAULT


def _retrieve(qb, p):
    sim = jax.lax.dot_general(
        qb, p, (((1,), (1,)), ((), ())), preferred_element_type=jnp.float32,
        precision=_DEF)
    m = jnp.max(sim, axis=-1, keepdims=True)
    e = jnp.exp((sim - m).astype(jnp.bfloat16))
    s = jnp.sum(e, axis=-1, keepdims=True, dtype=jnp.float32)
    num = jax.lax.dot_general(
        e, p, (((1,), (0,)), ((), ())),
        preferred_element_type=jnp.float32, precision=_DEF)
    return num * (1.0 / s)


def _body(qb_ref, pg_ref, pa_ref, pb_ref, w1_ref, b1_ref, w2t_ref, b2_ref,
          o_ref):
    qb = qb_ref[...].astype(jnp.bfloat16)
    rg = _retrieve(qb, pg_ref[...].astype(jnp.bfloat16))
    ra = _retrieve(qb, pa_ref[...].astype(jnp.bfloat16))
    rb = _retrieve(qb, pb_ref[...].astype(jnp.bfloat16))
    cr = 0.5 * (ra + rb)

    comb = jnp.concatenate([cr, rg], axis=-1)
    h = jax.lax.dot_general(
        comb.astype(jnp.bfloat16), w1_ref[...].astype(jnp.bfloat16),
        (((1,), (0,)), ((), ())),
        preferred_element_type=jnp.float32, precision=_DEF) + b1_ref[...]
    h = 0.5 * h * (1.0 + jax.lax.erf(h * 0.7071067811865476))
    hw = (h * w2t_ref[...]).astype(jnp.bfloat16)
    logit = jax.lax.dot_general(
        hw, jnp.ones((64, _D), jnp.bfloat16), (((1,), (0,)), ((), ())),
        preferred_element_type=jnp.float32, precision=_DEF) + b2_ref[...]
    gate = jax.nn.sigmoid(logit)
    o_ref[...] = gate * cr + (1.0 - gate) * rg


@functools.partial(jax.jit, static_argnames=())
def kernel(query, global_patterns, classA_patterns, classB_patterns,
           W1, b1, W2, b2):
    kg = global_patterns.shape[0]
    kc = classA_patterns.shape[0]
    grid = (_Q // _BQ,)
    out = pl.pallas_call(
        _body,
        grid=grid,
        in_specs=[
            pl.BlockSpec((_BQ, _D), lambda i: (i, 0)),
            pl.BlockSpec((kg, _D), lambda i: (0, 0)),
            pl.BlockSpec((kc, _D), lambda i: (0, 0)),
            pl.BlockSpec((kc, _D), lambda i: (0, 0)),
            pl.BlockSpec((2 * _D, 64), lambda i: (0, 0)),
            pl.BlockSpec((1, 64), lambda i: (0, 0)),
            pl.BlockSpec((1, 64), lambda i: (0, 0)),
            pl.BlockSpec((1, 1), lambda i: (0, 0)),
        ],
        out_specs=pl.BlockSpec((_BQ, _D), lambda i: (i, 0)),
        out_shape=jax.ShapeDtypeStruct((_Q, _D), jnp.float32),
    )(query, global_patterns, classA_patterns, classB_patterns,
      W1, b1.reshape(1, 64), W2.reshape(1, 64), b2.reshape(1, 1))
    return out
